# speedup vs baseline: 1.0517x; 1.0517x over previous
import functools

import jax
import jax.numpy as jnp
from jax import lax
from jax.experimental import pallas as pl
from jax.experimental.pallas import tpu as pltpu

N, D, H, O = 10000, 128, 128, 128
E = 320000
B, S, SEL = 10, 1000, 200

_INTERP = jax.default_backend() != "tpu"


def _node_mm_body(h_ref, wu_ref, wv_ref, ba_ref, u_ref, v_ref):
    h = h_ref[...]
    u_ref[...] = h @ wu_ref[...] + ba_ref[...]
    v_ref[...] = h @ wv_ref[...]


def _node_mm(h, Wa, ba, blk=2000):
    wu = Wa[:D] - Wa[D:]
    wv = Wa[D:]
    n = h.shape[0]
    return pl.pallas_call(
        _node_mm_body,
        grid=(n // blk,),
        in_specs=[
            pl.BlockSpec((blk, D), lambda i: (i, 0)),
            pl.BlockSpec((D, H), lambda i: (0, 0)),
            pl.BlockSpec((D, H), lambda i: (0, 0)),
            pl.BlockSpec((1, H), lambda i: (0, 0)),
        ],
        out_specs=[
            pl.BlockSpec((blk, H), lambda i: (i, 0)),
            pl.BlockSpec((blk, H), lambda i: (i, 0)),
        ],
        out_shape=[
            jax.ShapeDtypeStruct((n, H), jnp.float32),
            jax.ShapeDtypeStruct((n, H), jnp.float32),
        ],
        interpret=_INTERP,
    )(h, wu, wv, ba.reshape(1, H))


def _edge_mm_body(e_ref, w_ref, b_ref, o_ref):
    o_ref[...] = jnp.maximum(e_ref[...], 0.0) @ w_ref[...] + b_ref[...]


def _edge_mm(e_pre, Wb, bb, blk=2000):
    n = e_pre.shape[0]
    return pl.pallas_call(
        _edge_mm_body,
        grid=(n // blk,),
        in_specs=[
            pl.BlockSpec((blk, H), lambda i: (i, 0)),
            pl.BlockSpec((H, H), lambda i: (0, 0)),
            pl.BlockSpec((1, H), lambda i: (0, 0)),
        ],
        out_specs=pl.BlockSpec((blk, H), lambda i: (i, 0)),
        out_shape=jax.ShapeDtypeStruct((n, H), jnp.float32),
        interpret=_INTERP,
    )(e_pre, Wb, bb.reshape(1, H))


def _mlp_body(g_ref, w1_ref, b1_ref, w2_ref, b2_ref, o_ref):
    t = jnp.maximum(g_ref[...] @ w1_ref[...] + b1_ref[...], 0.0)
    o_ref[...] = t @ w2_ref[...] + b2_ref[...]


def _mlp(g, Wf1, bf1, Wf2, bf2, blk=2000):
    n = g.shape[0]
    return pl.pallas_call(
        _mlp_body,
        grid=(n // blk,),
        in_specs=[
            pl.BlockSpec((blk, H), lambda i: (i, 0)),
            pl.BlockSpec((H, H), lambda i: (0, 0)),
            pl.BlockSpec((1, H), lambda i: (0, 0)),
            pl.BlockSpec((H, O), lambda i: (0, 0)),
            pl.BlockSpec((1, O), lambda i: (0, 0)),
        ],
        out_specs=pl.BlockSpec((blk, O), lambda i: (i, 0)),
        out_shape=jax.ShapeDtypeStruct((n, O), jnp.float32),
        interpret=_INTERP,
    )(g, Wf1, bf1.reshape(1, H), Wf2, bf2.reshape(1, O))


def _layer(h, src, dst, Wa, ba, Wb, bb):
    u, v = _node_mm(h, Wa, ba)
    e_pre = u[dst] + v[src]
    m = _edge_mm(e_pre, Wb, bb)
    agg = jax.ops.segment_max(m, dst, num_segments=N)
    return jnp.maximum(agg, 0.0)


def kernel(x, edge_index, selected_idx, edge_label,
           W1a, b1a, W1b, b1b, W2a, b2a, W2b, b2b, W3a, b3a, W3b, b3b,
           Wf1, bf1, Wf2, bf2):
    src = edge_index[0]
    dst = edge_index[1]
    h = _layer(x, src, dst, W1a, b1a, W1b, b1b)
    h = _layer(h, src, dst, W2a, b2a, W2b, b2b)
    h = _layer(h, src, dst, W3a, b3a, W3b, b3b)

    flat_sel = (jnp.arange(B, dtype=jnp.int32)[:, None] * S + selected_idx).reshape(-1)
    rows = h[flat_sel]
    out = _mlp(rows, Wf1, bf1, Wf2, bf2, blk=2000)
    return out.reshape(B, SEL, O)


# speedup vs baseline: 1.3449x; 1.2787x over previous
import functools

import jax
import jax.numpy as jnp
from jax import lax
from jax.experimental import pallas as pl
from jax.experimental.pallas import tpu as pltpu
from jax.experimental.pallas import tpu_sc as plsc

N, D, H, O = 10000, 128, 128, 128
E = 320000
B, S, SEL = 10, 1000, 200

NW = 32
RNG = 320
AGGR = RNG + 1
CELL = 10240
SENT = RNG << 19
CHUNK = 32
GCH = 40
EPW = E // NW
SCAN = 2000
SELP = 2048

def _mesh():
    return plsc.VectorSubcoreMesh(core_axis_name="c", subcore_axis_name="s",
                                  num_cores=2, num_subcores=16)


def _wid():
    return lax.axis_index("s") * 2 + lax.axis_index("c")


def _take16(v, iv):
    dn = lax.GatherDimensionNumbers(offset_dims=(), collapsed_slice_dims=(0,),
                                    start_index_map=(0,))
    return lax.gather(v, iv[:, None], dn, (1,),
                      mode=lax.GatherScatterMode.PROMISE_IN_BOUNDS)


def _node_mm_body(h_ref, wu_ref, wv_ref, ba_ref, u_ref, v_ref):
    h = h_ref[...]
    u_ref[...] = h @ wu_ref[...] + ba_ref[...]
    v_ref[...] = h @ wv_ref[...]


def _node_mm(h, Wa, ba, blk=2000):
    wu = Wa[:D] - Wa[D:]
    wv = Wa[D:]
    n = h.shape[0]
    return pl.pallas_call(
        _node_mm_body,
        grid=(n // blk,),
        in_specs=[
            pl.BlockSpec((blk, D), lambda i: (i, 0)),
            pl.BlockSpec((D, H), lambda i: (0, 0)),
            pl.BlockSpec((D, H), lambda i: (0, 0)),
            pl.BlockSpec((1, H), lambda i: (0, 0)),
        ],
        out_specs=[
            pl.BlockSpec((blk, H), lambda i: (i, 0)),
            pl.BlockSpec((blk, H), lambda i: (i, 0)),
        ],
        out_shape=[
            jax.ShapeDtypeStruct((n, H), jnp.float32),
            jax.ShapeDtypeStruct((n, H), jnp.float32),
        ],
    )(h, wu, wv, ba.reshape(1, H))


def _edge_mm_body(e_ref, w_ref, b_ref, o_ref):
    o_ref[...] = jnp.maximum(e_ref[...], 0.0) @ w_ref[...] + b_ref[...]


def _edge_mm(e_pre, Wb, bb, blk=2000):
    n = e_pre.shape[0]
    return pl.pallas_call(
        _edge_mm_body,
        grid=(n // blk,),
        in_specs=[
            pl.BlockSpec((blk, H), lambda i: (i, 0)),
            pl.BlockSpec((H, H), lambda i: (0, 0)),
            pl.BlockSpec((1, H), lambda i: (0, 0)),
        ],
        out_specs=pl.BlockSpec((blk, H), lambda i: (i, 0)),
        out_shape=jax.ShapeDtypeStruct((n, H), jnp.float32),
    )(e_pre, Wb, bb.reshape(1, H))


def _mlp_body(g_ref, w1_ref, b1_ref, w2_ref, b2_ref, o_ref):
    t = jnp.maximum(g_ref[...] @ w1_ref[...] + b1_ref[...], 0.0)
    o_ref[...] = t @ w2_ref[...] + b2_ref[...]


def _mlp(g, Wf1, bf1, Wf2, bf2, blk=2048):
    n = g.shape[0]
    return pl.pallas_call(
        _mlp_body,
        grid=(n // blk,),
        in_specs=[
            pl.BlockSpec((blk, H), lambda i: (i, 0)),
            pl.BlockSpec((H, H), lambda i: (0, 0)),
            pl.BlockSpec((1, H), lambda i: (0, 0)),
            pl.BlockSpec((H, O), lambda i: (0, 0)),
            pl.BlockSpec((1, O), lambda i: (0, 0)),
        ],
        out_specs=pl.BlockSpec((blk, O), lambda i: (i, 0)),
        out_shape=jax.ShapeDtypeStruct((n, O), jnp.float32),
    )(g, Wf1, bf1.reshape(1, H), Wf2, bf2.reshape(1, O))


def _bin_body(dst_hbm, li_hbm, ld_hbm, cnts_hbm, dbuf, obi, obd, cbuf, sem):
    wid = _wid()
    base = wid * EPW
    iota = lax.iota(jnp.int32, 16)
    zeros16 = jnp.zeros((16,), jnp.int32)
    sentd = jnp.full((16,), RNG, jnp.int32)
    lane15 = jnp.full((16,), 15, jnp.int32)
    pltpu.sync_copy(dst_hbm.at[pl.ds(base, EPW)], dbuf)

    def range_body(r, carry):
        lo = r * RNG

        def g_body(g, off16):
            d16 = dbuf[pl.ds(g * 16, 16)]
            msk = (d16 >= lo) & (d16 < lo + RNG)
            pos = plsc.cumsum(msk.astype(jnp.int32))
            idx = off16 + pos - 1
            plsc.store_scatter(obi, [idx], base + g * 16 + iota, mask=msk)
            plsc.store_scatter(obd, [idx], d16 - lo, mask=msk)
            return off16 + _take16(pos, lane15)

        off16 = lax.fori_loop(0, EPW // 16, g_body,
                              jnp.zeros((16,), jnp.int32))
        for k in range(CHUNK // 16):
            plsc.store_scatter(obi, [off16 + (iota + k * 16)], zeros16)
            plsc.store_scatter(obd, [off16 + (iota + k * 16)], sentd)
        cbuf[...] = off16
        cell = (r * NW + wid) * CELL
        pltpu.sync_copy(obi, li_hbm.at[pl.ds(cell, CELL)])
        pltpu.sync_copy(obd, ld_hbm.at[pl.ds(cell, CELL)])
        pltpu.sync_copy(cbuf, cnts_hbm.at[pl.ds((r * NW + wid) * 16, 16)])
        return carry

    lax.fori_loop(0, NW, range_body, 0)


def _bin_edges(dst):
    k = pl.kernel(
        _bin_body,
        out_type=[
            jax.ShapeDtypeStruct((NW * NW * CELL,), jnp.int32),
            jax.ShapeDtypeStruct((NW * NW * CELL,), jnp.int32),
            jax.ShapeDtypeStruct((NW * NW * 16,), jnp.int32),
        ],
        mesh=_mesh(),
        compiler_params=pltpu.CompilerParams(needs_layout_passes=False),
        scratch_types=[
            pltpu.VMEM((EPW,), jnp.int32),
            pltpu.VMEM((CELL,), jnp.int32),
            pltpu.VMEM((CELL,), jnp.int32),
            pltpu.VMEM((16,), jnp.int32),
            pltpu.SemaphoreType.DMA,
        ],
    )
    return k(dst)


def _gather_body(u_hbm, v_hbm, src_hbm, dst_hbm, out_hbm,
                 dbuf, sbuf, urows, vrows, sem):
    wid = _wid()
    base = wid * EPW

    def ch_body(c, carry):
        off = base + c * GCH
        pltpu.sync_copy(dst_hbm.at[pl.ds(off, GCH)], dbuf)
        pltpu.sync_copy(src_hbm.at[pl.ds(off, GCH)], sbuf)
        pltpu.async_copy(u_hbm.at[dbuf], urows, sem).wait()
        pltpu.async_copy(v_hbm.at[sbuf], vrows, sem).wait()

        def g_body(g, carry):
            for j in range(H // 16):
                s = pl.ds(j * 16, 16)
                urows[g, s] = urows[g, s] + vrows[g, s]
            return carry

        lax.fori_loop(0, GCH, g_body, 0)
        pltpu.sync_copy(urows, out_hbm.at[pl.ds(off, GCH)])
        return carry

    lax.fori_loop(0, EPW // GCH, ch_body, 0)


def _edge_gather(u, v, src, dst):
    k = pl.kernel(
        _gather_body,
        out_type=jax.ShapeDtypeStruct((E, H), jnp.float32),
        mesh=_mesh(),
        compiler_params=pltpu.CompilerParams(needs_layout_passes=False),
        scratch_types=[
            pltpu.VMEM((GCH,), jnp.int32),
            pltpu.VMEM((GCH,), jnp.int32),
            pltpu.VMEM((GCH, H), jnp.float32),
            pltpu.VMEM((GCH, H), jnp.float32),
            pltpu.SemaphoreType.DMA,
        ],
    )
    return k(u, v, src, dst)


def _smax_body(m_hbm, li_hbm, ld_hbm, cnts_hbm, out_hbm,
               ibuf, lbuf, mrows, agg, cbuf, sem):
    wid = _wid()
    iota = lax.iota(jnp.int32, 16)
    zero = jnp.zeros((16,), jnp.float32)

    def z_body(i, carry):
        for j in range(H // 16):
            agg[i, pl.ds(j * 16, 16)] = zero
        return carry

    lax.fori_loop(0, AGGR, z_body, 0)

    pltpu.sync_copy(cnts_hbm.at[pl.ds(wid * NW * 16, NW * 16)], cbuf)
    cols = [iota + (j * 16) for j in range(H // 16)]

    def w_body(w2, carry):
        cell = (wid * NW + w2) * CELL
        cnt = jnp.max(cbuf[pl.ds(w2 * 16, 16)])
        nch = (cnt + CHUNK - 1) // CHUNK

        def ch_body(ch, carry):
            pltpu.sync_copy(li_hbm.at[pl.ds(cell + ch * CHUNK, CHUNK)], ibuf)
            pltpu.sync_copy(ld_hbm.at[pl.ds(cell + ch * CHUNK, CHUNK)], lbuf)
            pltpu.async_copy(m_hbm.at[ibuf], mrows, sem).wait()

            def g_body(g, carry):
                d16 = lbuf[pl.ds(g * 16, 16)]
                for e in range(16):
                    dsp = _take16(d16, jnp.full((16,), e, jnp.int32))
                    vals = [mrows[g * 16 + e, pl.ds(j * 16, 16)]
                            for j in range(H // 16)]
                    olds = [plsc.load_gather(agg, [dsp, cols[j]])
                            for j in range(H // 16)]
                    for j in range(H // 16):
                        plsc.store_scatter(agg, [dsp, cols[j]],
                                           jnp.maximum(olds[j], vals[j]))
                return carry

            lax.fori_loop(0, CHUNK // 16, g_body, 0)
            return carry

        lax.fori_loop(0, nch, ch_body, 0)
        return carry

    lax.fori_loop(0, NW, w_body, 0)
    pltpu.sync_copy(agg.at[pl.ds(0, RNG)], out_hbm.at[pl.ds(wid * RNG, RNG)])


def _scatter_max(m, lists_i, lists_d, cnts):
    k = pl.kernel(
        _smax_body,
        out_type=jax.ShapeDtypeStruct((NW * RNG, H), jnp.float32),
        mesh=_mesh(),
        compiler_params=pltpu.CompilerParams(needs_layout_passes=False),
        scratch_types=[
            pltpu.VMEM((CHUNK,), jnp.int32),
            pltpu.VMEM((CHUNK,), jnp.int32),
            pltpu.VMEM((CHUNK, H), jnp.float32),
            pltpu.VMEM((AGGR, H), jnp.float32),
            pltpu.VMEM((NW * 16,), jnp.int32),
            pltpu.SemaphoreType.DMA,
        ],
    )
    return k(m, lists_i, lists_d, cnts)


def _sel_body(h_hbm, idx_hbm, out_hbm, ibuf, rows, sem):
    wid = _wid()
    per = SELP // NW
    base = wid * per
    pltpu.sync_copy(idx_hbm.at[pl.ds(base, per)], ibuf)
    pltpu.async_copy(h_hbm.at[ibuf], rows, sem).wait()
    pltpu.sync_copy(rows, out_hbm.at[pl.ds(base, per)])


def _sel_gather(h, flat_sel):
    k = pl.kernel(
        _sel_body,
        out_type=jax.ShapeDtypeStruct((SELP, H), jnp.float32),
        mesh=_mesh(),
        compiler_params=pltpu.CompilerParams(needs_layout_passes=False),
        scratch_types=[
            pltpu.VMEM((SELP // NW,), jnp.int32),
            pltpu.VMEM((SELP // NW, H), jnp.float32),
            pltpu.SemaphoreType.DMA,
        ],
    )
    return k(h, flat_sel)


def _layer(h, src, dst, lists_i, lists_d, cnts, Wa, ba, Wb, bb):
    u, v = _node_mm(h, Wa, ba)
    e_pre = _edge_gather(u, v, src, dst)
    m = _edge_mm(e_pre, Wb, bb)
    agg = _scatter_max(m, lists_i, lists_d, cnts)
    return agg[:N]


def kernel(x, edge_index, selected_idx, edge_label,
           W1a, b1a, W1b, b1b, W2a, b2a, W2b, b2b, W3a, b3a, W3b, b3b,
           Wf1, bf1, Wf2, bf2):
    src = edge_index[0]
    dst = edge_index[1]
    lists_i, lists_d, cnts = _bin_edges(dst)
    h = _layer(x, src, dst, lists_i, lists_d, cnts, W1a, b1a, W1b, b1b)
    h = _layer(h, src, dst, lists_i, lists_d, cnts, W2a, b2a, W2b, b2b)
    h = _layer(h, src, dst, lists_i, lists_d, cnts, W3a, b3a, W3b, b3b)

    flat_sel = (jnp.arange(B, dtype=jnp.int32)[:, None] * S
                + selected_idx).reshape(-1)
    flat_sel = jnp.concatenate(
        [flat_sel, jnp.zeros((SELP - B * SEL,), jnp.int32)])
    rows = _sel_gather(h, flat_sel)
    out = _mlp(rows, Wf1, bf1, Wf2, bf2)
    return out[:B * SEL].reshape(B, SEL, O)
